# Initial kernel scaffold; baseline (speedup 1.0000x reference)
import jax
import jax.numpy as jnp
from jax.experimental import pallas as pl


def kernel(x, edge_index, W1, b1, W2, b2, W3, b3, in_w, in_b, out_w, out_b, ln1_g, ln1_b, ln2_g, ln2_b, mw1, mb1, mw2, mb2, lw, lb):
    raise NotImplementedError("write your pallas kernel here")


# speedup vs baseline: 8.2572x; 8.2572x over previous
import functools
import jax
import jax.numpy as jnp
import numpy as np
from jax import lax
from jax.experimental import pallas as pl
from jax.experimental.pallas import tpu as pltpu
from jax.experimental.pallas import tpu_sc as plsc

_T, _N, _E, _D, _NH, _FF, _C = 8, 10000, 320000, 128, 4, 512, 10
_NSC = 2
_NSUB = 16
_CH = 128
_G = 8
_NSUP = 20
_CPT = _NSUP * _G
_EPT = _CPT * _CH
_EPAD = _NSUB * _EPT
_NACC = 10240
_ZROWS = 640
_DROWS = _N // _NSUB
_TSC = _T // _NSC


def _process_super(table, acc, sidx, didx, rows0, rows1, g0, g1, s0, s1):
    pltpu.async_copy(table.at[sidx.at[0]], rows0, g0)
    pltpu.async_copy(table.at[sidx.at[1]], rows1, g1)

    def pair(jj, _):
        j0 = 2 * jj
        j1 = j0 + 1
        pltpu.make_async_copy(table.at[sidx.at[j0]], rows0, g0).wait()
        pltpu.async_copy(rows0, acc.at[didx.at[j0]], s0, add=True)
        pltpu.make_async_copy(table.at[sidx.at[j1]], rows1, g1).wait()
        pltpu.make_async_copy(rows0, acc.at[didx.at[j0]], s0).wait()

        @pl.when(j0 + 2 < _G)
        def _():
            pltpu.async_copy(table.at[sidx.at[j0 + 2]], rows0, g0)

        pltpu.async_copy(rows1, acc.at[didx.at[j1]], s1, add=True)
        pltpu.make_async_copy(rows1, acc.at[didx.at[j1]], s1).wait()

        @pl.when(j1 + 2 < _G)
        def _():
            pltpu.async_copy(table.at[sidx.at[j1 + 2]], rows1, g1)
        return 0

    lax.fori_loop(0, _G // 2, pair, 0)


def _segsum_body(table, srcr, dstr, zeros_hbm, out, sidx0, sidx1, didx0, didx1,
                 rows0, rows1, acc, g0, g1, s0, s1, i0, i1):
    c = lax.axis_index("c")
    s = lax.axis_index("s")

    for tt in range(_TSC):
        t = c * _TSC + tt
        pltpu.sync_copy(zeros_hbm, acc.at[pl.ds(s * _ZROWS, _ZROWS)])
        pltpu.sync_copy(srcr.at[t, s, 0], sidx0)
        pltpu.sync_copy(dstr.at[t, s, 0], didx0)
        pltpu.async_copy(srcr.at[t, s, 1], sidx1, i1)
        pltpu.async_copy(dstr.at[t, s, 1], didx1, i1)
        plsc.subcore_barrier()

        def gpair(gg, _):
            ge = 2 * gg
            _process_super(table, acc, sidx0, didx0, rows0, rows1,
                           g0, g1, s0, s1)

            @pl.when(ge + 2 < _NSUP)
            def _():
                pltpu.async_copy(srcr.at[t, s, ge + 2], sidx0, i0)
                pltpu.async_copy(dstr.at[t, s, ge + 2], didx0, i0)

            pltpu.make_async_copy(srcr.at[t, s, ge + 1], sidx1, i1).wait()
            pltpu.make_async_copy(dstr.at[t, s, ge + 1], didx1, i1).wait()
            _process_super(table, acc, sidx1, didx1, rows0, rows1,
                           g0, g1, s0, s1)

            @pl.when(ge + 3 < _NSUP)
            def _():
                pltpu.async_copy(srcr.at[t, s, ge + 3], sidx1, i1)
                pltpu.async_copy(dstr.at[t, s, ge + 3], didx1, i1)

            @pl.when(ge + 2 < _NSUP)
            def _():
                pltpu.make_async_copy(srcr.at[t, s, ge + 2], sidx0, i0).wait()
                pltpu.make_async_copy(dstr.at[t, s, ge + 2], didx0, i0).wait()
            return 0

        lax.fori_loop(0, _NSUP // 2, gpair, 0)
        plsc.subcore_barrier()
        pltpu.sync_copy(acc.at[pl.ds(s * _ZROWS, _ZROWS)],
                        out.at[t, pl.ds(s * _ZROWS, _ZROWS)])
        plsc.subcore_barrier()


def _sc_segsum(table_flat, srcr, dstr, zeros_hbm):
    mesh = plsc.VectorSubcoreMesh(core_axis_name="c", subcore_axis_name="s")
    return pl.kernel(
        _segsum_body,
        out_type=jax.ShapeDtypeStruct((_T, _NACC, _D), jnp.float32),
        mesh=mesh,
        scratch_types=[
            pltpu.VMEM((_G, _CH), jnp.int32),
            pltpu.VMEM((_G, _CH), jnp.int32),
            pltpu.VMEM((_G, _CH), jnp.int32),
            pltpu.VMEM((_G, _CH), jnp.int32),
            pltpu.VMEM((_CH, _D), jnp.float32),
            pltpu.VMEM((_CH, _D), jnp.float32),
            pltpu.VMEM_SHARED((_NACC, _D), jnp.float32),
            pltpu.SemaphoreType.DMA,
            pltpu.SemaphoreType.DMA,
            pltpu.SemaphoreType.DMA,
            pltpu.SemaphoreType.DMA,
            pltpu.SemaphoreType.DMA,
            pltpu.SemaphoreType.DMA,
        ],
    )(table_flat, srcr, dstr, zeros_hbm)


def _deg_body(dstr, ones_hbm, zeros_hbm, out, didx, ones_v, acc, ssem):
    c = lax.axis_index("c")
    s = lax.axis_index("s")
    pltpu.sync_copy(ones_hbm, ones_v)
    for tt in range(_TSC):
        t = c * _TSC + tt
        pltpu.sync_copy(zeros_hbm, acc.at[pl.ds(s * _ZROWS, _ZROWS)])
        plsc.subcore_barrier()

        def super_step(g, _):
            pltpu.sync_copy(dstr.at[t, s, g], didx)
            for j in range(_G):
                pltpu.async_copy(ones_v, acc.at[didx.at[j]], ssem, add=True)
            for j in range(_G):
                pltpu.make_async_copy(ones_v, acc.at[didx.at[j]],
                                      ssem).wait()
            return 0

        lax.fori_loop(0, _NSUP, super_step, 0)
        plsc.subcore_barrier()
        pltpu.sync_copy(acc.at[pl.ds(s * _ZROWS, _ZROWS)],
                        out.at[t, pl.ds(s * _ZROWS, _ZROWS)])
        plsc.subcore_barrier()


def _sc_deg(dstr, ones_hbm, zeros_hbm):
    mesh = plsc.VectorSubcoreMesh(core_axis_name="c", subcore_axis_name="s")
    return pl.kernel(
        _deg_body,
        out_type=jax.ShapeDtypeStruct((_T, _NACC, _D), jnp.float32),
        mesh=mesh,
        scratch_types=[
            pltpu.VMEM((_G, _CH), jnp.int32),
            pltpu.VMEM((_CH, _D), jnp.float32),
            pltpu.VMEM_SHARED((_NACC, _D), jnp.float32),
            pltpu.SemaphoreType.DMA,
        ],
    )(dstr, ones_hbm, zeros_hbm)


_BR = 1000


def _prescale_body(x_ref, w_ref, deg_ref, out_ref, dis_ref):
    dis = lax.rsqrt(deg_ref[0, :, 0:1] + 1.0)
    out_ref[0] = dis * jnp.dot(x_ref[0], w_ref[...],
                               preferred_element_type=jnp.float32)
    dis_ref[0] = jnp.broadcast_to(dis, (_BR, 16))


def _tc_prescale(x, w, deg):
    return pl.pallas_call(
        _prescale_body,
        out_shape=[
            jax.ShapeDtypeStruct((_T, _N, _D), jnp.float32),
            jax.ShapeDtypeStruct((_T, _N, 16), jnp.float32),
        ],
        grid=(_T, _N // _BR),
        in_specs=[
            pl.BlockSpec((1, _BR, _D), lambda t, j: (t, j, 0)),
            pl.BlockSpec((_D, _D), lambda t, j: (0, 0)),
            pl.BlockSpec((1, _BR, _D), lambda t, j: (t, j, 0)),
        ],
        out_specs=[
            pl.BlockSpec((1, _BR, _D), lambda t, j: (t, j, 0)),
            pl.BlockSpec((1, _BR, 16), lambda t, j: (t, j, 0)),
        ],
    )(x, w, deg)


def _layer_body(acc_ref, yw_ref, dis16_ref, w_ref, b_ref, out_ref):
    dis = dis16_ref[0, :, 0:1]
    h = jnp.tanh(dis * (acc_ref[0] + yw_ref[0]) + b_ref[...])
    out_ref[0] = dis * jnp.dot(h, w_ref[...],
                               preferred_element_type=jnp.float32)


def _tc_layer(acc, yw, dis16, w_next, b_prev):
    return pl.pallas_call(
        _layer_body,
        out_shape=jax.ShapeDtypeStruct((_T, _N, _D), jnp.float32),
        grid=(_T, _N // _BR),
        in_specs=[
            pl.BlockSpec((1, _BR, _D), lambda t, j: (t, j, 0)),
            pl.BlockSpec((1, _BR, _D), lambda t, j: (t, j, 0)),
            pl.BlockSpec((1, _BR, 16), lambda t, j: (t, j, 0)),
            pl.BlockSpec((_D, _D), lambda t, j: (0, 0)),
            pl.BlockSpec((1, _D), lambda t, j: (0, 0)),
        ],
        out_specs=pl.BlockSpec((1, _BR, _D), lambda t, j: (t, j, 0)),
    )(acc, yw, dis16, w_next, b_prev)


def _final_body(acc_ref, yw_ref, dis16_ref, b_ref, out_ref):
    t = pl.program_id(0)
    j = pl.program_id(1)
    dis = dis16_ref[0, :, 0:1]
    h = jnp.tanh(dis * (acc_ref[0] + yw_ref[0]) + b_ref[...])
    part = jnp.sum(h, axis=0, keepdims=True)

    @pl.when(j == 0)
    def _():
        out_ref[pl.ds(t, 1), :] = part

    @pl.when(j > 0)
    def _():
        out_ref[pl.ds(t, 1), :] = out_ref[pl.ds(t, 1), :] + part


def _tc_final(acc, yw, dis16, b3):
    return pl.pallas_call(
        _final_body,
        out_shape=jax.ShapeDtypeStruct((_T, _D), jnp.float32),
        grid=(_T, _N // _BR),
        in_specs=[
            pl.BlockSpec((1, _BR, _D), lambda t, j: (t, j, 0)),
            pl.BlockSpec((1, _BR, _D), lambda t, j: (t, j, 0)),
            pl.BlockSpec((1, _BR, 16), lambda t, j: (t, j, 0)),
            pl.BlockSpec((1, _D), lambda t, j: (0, 0)),
        ],
        out_specs=pl.BlockSpec((_T, _D), lambda t, j: (0, 0)),
    )(acc, yw, dis16, b3)


def _head_body(seq_ref, inw_ref, inb_ref, outw_ref, outb_ref, ln2g_ref,
               ln2b_ref, mw1_ref, mb1_ref, mw2_ref, mb2_ref, lw_ref, lb_ref,
               out_ref):
    seq = seq_ref[...]
    inw = inw_ref[...]
    inb = inb_ref[...]

    def proj(lo):
        return (lax.dot_general(seq, inw[lo:lo + _D, :],
                                (((1,), (1,)), ((), ())),
                                preferred_element_type=jnp.float32)
                + inb[0:1, lo:lo + _D])

    q = proj(0)
    k = proj(_D)
    v = proj(2 * _D)
    hd = _D // _NH
    scale = np.float32(1.0 / np.sqrt(hd).astype(np.float32))
    parts = []
    for h in range(_NH):
        qh = q[:, h * hd:(h + 1) * hd]
        kh = k[:, h * hd:(h + 1) * hd]
        vh = v[:, h * hd:(h + 1) * hd]
        sc = lax.dot_general(qh, kh, (((1,), (1,)), ((), ())),
                             preferred_element_type=jnp.float32) * scale
        m = jnp.max(sc, axis=-1, keepdims=True)
        e = jnp.exp(sc - m)
        attn = e / jnp.sum(e, axis=-1, keepdims=True)
        parts.append(jnp.dot(attn, vh, preferred_element_type=jnp.float32))
    ao = jnp.concatenate(parts, axis=1)
    xa = lax.dot_general(ao, outw_ref[...], (((1,), (1,)), ((), ())),
                         preferred_element_type=jnp.float32) + outb_ref[...]
    h2 = jax.nn.relu(
        lax.dot_general(xa, mw1_ref[...], (((1,), (1,)), ((), ())),
                        preferred_element_type=jnp.float32) + mb1_ref[...])
    h2 = lax.dot_general(h2, mw2_ref[...], (((1,), (1,)), ((), ())),
                         preferred_element_type=jnp.float32) + mb2_ref[...]
    xa = xa + h2
    mean = jnp.mean(xa, axis=-1, keepdims=True)
    var = jnp.mean((xa - mean) ** 2, axis=-1, keepdims=True)
    xa = (xa - mean) / jnp.sqrt(var + 1e-5) * ln2g_ref[...] + ln2b_ref[...]
    xr = jax.nn.relu(xa)
    pooled = jnp.sum(xr, axis=0, keepdims=True)
    out_ref[...] = (lax.dot_general(pooled, lw_ref[...],
                                    (((1,), (1,)), ((), ())),
                                    preferred_element_type=jnp.float32)
                    + lb_ref[...])


def _tc_head(seq, in_w, in_b, out_w, out_b, ln2_g, ln2_b, mw1, mb1, mw2, mb2,
             lw_pad, lb_pad):
    return pl.pallas_call(
        _head_body,
        out_shape=jax.ShapeDtypeStruct((1, _D), jnp.float32),
    )(seq, in_w, in_b, out_w, out_b, ln2_g, ln2_b, mw1, mb1, mw2, mb2,
      lw_pad, lb_pad)


@jax.jit
def kernel(x, edge_index, W1, b1, W2, b2, W3, b3, in_w, in_b, out_w, out_b,
           ln1_g, ln1_b, ln2_g, ln2_b, mw1, mb1, mw2, mb2, lw, lb):
    src = edge_index[:, 0, :]
    dst = edge_index[:, 1, :]
    pad = _EPAD - _E
    src_pad = jnp.concatenate(
        [src, jnp.zeros((_T, pad), jnp.int32)], axis=1)
    dst_pad = jnp.concatenate(
        [dst, jnp.full((_T, pad), _N, jnp.int32)], axis=1)
    src_off = src_pad + (jnp.arange(_T, dtype=jnp.int32) * _N)[:, None]
    srcr = src_off.reshape(_T, _NSUB, _NSUP, _G, _CH)
    dstr = dst_pad.reshape(_T, _NSUB, _NSUP, _G, _CH)

    zeros128 = jnp.zeros((_ZROWS, _D), jnp.float32)
    ones128 = jnp.ones((_CH, _D), jnp.float32)

    deg = _sc_deg(dstr, ones128, zeros128)

    yw1, dis16 = _tc_prescale(x, W1, deg)
    acc1 = _sc_segsum(yw1.reshape(_T * _N, _D), srcr, dstr, zeros128)
    yw2 = _tc_layer(acc1, yw1, dis16, W2, b1.reshape(1, _D))
    acc2 = _sc_segsum(yw2.reshape(_T * _N, _D), srcr, dstr, zeros128)
    yw3 = _tc_layer(acc2, yw2, dis16, W3, b2.reshape(1, _D))
    acc3 = _sc_segsum(yw3.reshape(_T * _N, _D), srcr, dstr, zeros128)
    seq = _tc_final(acc3, yw3, dis16, b3.reshape(1, _D))

    lw_pad = jnp.zeros((_D, _D), jnp.float32).at[: _C, :].set(lw)
    lb_pad = jnp.zeros((1, _D), jnp.float32).at[0, : _C].set(lb)
    out = _tc_head(seq, in_w, in_b.reshape(1, 3 * _D), out_w,
                   out_b.reshape(1, _D), ln2_g.reshape(1, _D),
                   ln2_b.reshape(1, _D), mw1, mb1.reshape(1, _FF), mw2,
                   mb2.reshape(1, _D), lw_pad, lb_pad)
    return out[0, : _C]


# speedup vs baseline: 8.9779x; 1.0873x over previous
import functools
import jax
import jax.numpy as jnp
import numpy as np
from jax import lax
from jax.experimental import pallas as pl
from jax.experimental.pallas import tpu as pltpu
from jax.experimental.pallas import tpu_sc as plsc

_T, _N, _E, _D, _NH, _FF, _C = 8, 10000, 320000, 128, 4, 512, 10
_NSC = 2
_NSUB = 16
_CH = 64
_G = 10
_NSUP = 32
_CPT = _NSUP * _G
_EPT = _CPT * _CH
_EPAD = _NSUB * _EPT
_NACC = 10112
_ZROWS = 632
_TSC = _T // _NSC
_B = 5
_A = 3


def _segsum_body(table, srcr, dstr, zeros_hbm, out, sidx0, sidx1, didx0,
                 didx1, r0, r1, r2, r3, r4, acc, g0, g1, g2, g3, g4,
                 s0, s1, s2, s3, s4, i0, i1):
    rows = [r0, r1, r2, r3, r4]
    gsem = [g0, g1, g2, g3, g4]
    ssem = [s0, s1, s2, s3, s4]
    isem = [i0, i1]
    c = lax.axis_index("c")
    sid = lax.axis_index("s")

    def wait_g(b):
        pltpu.make_async_copy(table.at[sidx0.at[0]], rows[b], gsem[b]).wait()

    def wait_s(b):
        pltpu.make_async_copy(rows[b], acc.at[didx0.at[0]], ssem[b]).wait()

    for tt in range(_TSC):
        t = c * _TSC + tt
        pltpu.sync_copy(zeros_hbm, acc.at[pl.ds(sid * _ZROWS, _ZROWS)])
        pltpu.sync_copy(srcr.at[t, sid, 0], sidx0)
        pltpu.sync_copy(dstr.at[t, sid, 0], didx0)
        pltpu.async_copy(srcr.at[t, sid, 1], sidx1, i1)
        pltpu.async_copy(dstr.at[t, sid, 1], didx1, i1)
        plsc.subcore_barrier()

        for p in range(_A):
            pltpu.async_copy(table.at[sidx0.at[p]], rows[p], gsem[p])

        def do_super(g, sidx_cur, didx_cur, sidx_nxt, didx_nxt, i_nxt):
            for p in range(_G):
                b = p % _B
                j = g * _G + p
                wait_g(b)
                pltpu.async_copy(rows[b], acc.at[didx_cur.at[p]], ssem[b],
                                 add=True)

                @pl.when(j >= 2)
                def _():
                    wait_s((b - 2) % _B)

                if p == _G - _A:
                    @pl.when(g + 1 < _NSUP)
                    def _():
                        pltpu.make_async_copy(srcr.at[t, sid, 0], sidx_nxt,
                                              i_nxt).wait()
                        pltpu.make_async_copy(dstr.at[t, sid, 0], didx_nxt,
                                              i_nxt).wait()

                bq = (b + _A) % _B
                tgt = p + _A

                @pl.when(j + _A < _CPT)
                def _():
                    if tgt < _G:
                        pltpu.async_copy(table.at[sidx_cur.at[tgt]], rows[bq],
                                         gsem[bq])
                    else:
                        pltpu.async_copy(table.at[sidx_nxt.at[tgt - _G]],
                                         rows[bq], gsem[bq])

            @pl.when(g + 2 < _NSUP)
            def _():
                pltpu.async_copy(srcr.at[t, sid, g + 2], sidx_cur,
                                 isem[0] if sidx_cur is sidx0 else isem[1])
                pltpu.async_copy(dstr.at[t, sid, g + 2], didx_cur,
                                 isem[0] if sidx_cur is sidx0 else isem[1])

        def super_pair(gg, _):
            do_super(2 * gg, sidx0, didx0, sidx1, didx1, i1)
            do_super(2 * gg + 1, sidx1, didx1, sidx0, didx0, i0)
            return 0

        lax.fori_loop(0, _NSUP // 2, super_pair, 0)
        wait_s((_CPT - 2) % _B)
        wait_s((_CPT - 1) % _B)
        plsc.subcore_barrier()
        pltpu.sync_copy(acc.at[pl.ds(sid * _ZROWS, _ZROWS)],
                        out.at[t, pl.ds(sid * _ZROWS, _ZROWS)])
        plsc.subcore_barrier()


def _sc_segsum(table_flat, srcr, dstr, zeros_hbm):
    mesh = plsc.VectorSubcoreMesh(core_axis_name="c", subcore_axis_name="s")
    return pl.kernel(
        _segsum_body,
        out_type=jax.ShapeDtypeStruct((_T, _NACC, _D), jnp.float32),
        mesh=mesh,
        scratch_types=(
            [pltpu.VMEM((_G, _CH), jnp.int32) for _ in range(4)]
            + [pltpu.VMEM((_CH, _D), jnp.float32) for _ in range(_B)]
            + [pltpu.VMEM_SHARED((_NACC, _D), jnp.float32)]
            + [pltpu.SemaphoreType.DMA for _ in range(2 * _B + 2)]
        ),
    )(table_flat, srcr, dstr, zeros_hbm)


def _deg_body(dstr, ones_hbm, zeros_hbm, out, didx, ones_v, acc, ssem):
    c = lax.axis_index("c")
    s = lax.axis_index("s")
    pltpu.sync_copy(ones_hbm, ones_v)
    for tt in range(_TSC):
        t = c * _TSC + tt
        pltpu.sync_copy(zeros_hbm, acc.at[pl.ds(s * _ZROWS, _ZROWS)])
        plsc.subcore_barrier()

        def super_step(g, _):
            pltpu.sync_copy(dstr.at[t, s, g], didx)
            for j in range(_G):
                pltpu.async_copy(ones_v, acc.at[didx.at[j]], ssem, add=True)
            for j in range(_G):
                pltpu.make_async_copy(ones_v, acc.at[didx.at[j]],
                                      ssem).wait()
            return 0

        lax.fori_loop(0, _NSUP, super_step, 0)
        plsc.subcore_barrier()
        pltpu.sync_copy(acc.at[pl.ds(s * _ZROWS, _ZROWS)],
                        out.at[t, pl.ds(s * _ZROWS, _ZROWS)])
        plsc.subcore_barrier()


def _sc_deg(dstr, ones_hbm, zeros_hbm):
    mesh = plsc.VectorSubcoreMesh(core_axis_name="c", subcore_axis_name="s")
    return pl.kernel(
        _deg_body,
        out_type=jax.ShapeDtypeStruct((_T, _NACC, _D), jnp.float32),
        mesh=mesh,
        scratch_types=[
            pltpu.VMEM((_G, _CH), jnp.int32),
            pltpu.VMEM((_CH, _D), jnp.float32),
            pltpu.VMEM_SHARED((_NACC, _D), jnp.float32),
            pltpu.SemaphoreType.DMA,
        ],
    )(dstr, ones_hbm, zeros_hbm)


_BR = 1000


def _prescale_body(x_ref, w_ref, deg_ref, out_ref, dis_ref):
    dis = lax.rsqrt(deg_ref[0, :, 0:1] + 1.0)
    out_ref[0] = dis * jnp.dot(x_ref[0], w_ref[...],
                               preferred_element_type=jnp.float32)
    dis_ref[0] = jnp.broadcast_to(dis, (_BR, 16))


def _tc_prescale(x, w, deg):
    return pl.pallas_call(
        _prescale_body,
        out_shape=[
            jax.ShapeDtypeStruct((_T, _N, _D), jnp.float32),
            jax.ShapeDtypeStruct((_T, _N, 16), jnp.float32),
        ],
        grid=(_T, _N // _BR),
        in_specs=[
            pl.BlockSpec((1, _BR, _D), lambda t, j: (t, j, 0)),
            pl.BlockSpec((_D, _D), lambda t, j: (0, 0)),
            pl.BlockSpec((1, _BR, _D), lambda t, j: (t, j, 0)),
        ],
        out_specs=[
            pl.BlockSpec((1, _BR, _D), lambda t, j: (t, j, 0)),
            pl.BlockSpec((1, _BR, 16), lambda t, j: (t, j, 0)),
        ],
    )(x, w, deg)


def _layer_body(acc_ref, yw_ref, dis16_ref, w_ref, b_ref, out_ref):
    dis = dis16_ref[0, :, 0:1]
    h = jnp.tanh(dis * (acc_ref[0] + yw_ref[0]) + b_ref[...])
    out_ref[0] = dis * jnp.dot(h, w_ref[...],
                               preferred_element_type=jnp.float32)


def _tc_layer(acc, yw, dis16, w_next, b_prev):
    return pl.pallas_call(
        _layer_body,
        out_shape=jax.ShapeDtypeStruct((_T, _N, _D), jnp.float32),
        grid=(_T, _N // _BR),
        in_specs=[
            pl.BlockSpec((1, _BR, _D), lambda t, j: (t, j, 0)),
            pl.BlockSpec((1, _BR, _D), lambda t, j: (t, j, 0)),
            pl.BlockSpec((1, _BR, 16), lambda t, j: (t, j, 0)),
            pl.BlockSpec((_D, _D), lambda t, j: (0, 0)),
            pl.BlockSpec((1, _D), lambda t, j: (0, 0)),
        ],
        out_specs=pl.BlockSpec((1, _BR, _D), lambda t, j: (t, j, 0)),
    )(acc, yw, dis16, w_next, b_prev)


def _final_body(acc_ref, yw_ref, dis16_ref, b_ref, out_ref):
    t = pl.program_id(0)
    j = pl.program_id(1)
    dis = dis16_ref[0, :, 0:1]
    h = jnp.tanh(dis * (acc_ref[0] + yw_ref[0]) + b_ref[...])
    part = jnp.sum(h, axis=0, keepdims=True)

    @pl.when(j == 0)
    def _():
        out_ref[pl.ds(t, 1), :] = part

    @pl.when(j > 0)
    def _():
        out_ref[pl.ds(t, 1), :] = out_ref[pl.ds(t, 1), :] + part


def _tc_final(acc, yw, dis16, b3):
    return pl.pallas_call(
        _final_body,
        out_shape=jax.ShapeDtypeStruct((_T, _D), jnp.float32),
        grid=(_T, _N // _BR),
        in_specs=[
            pl.BlockSpec((1, _BR, _D), lambda t, j: (t, j, 0)),
            pl.BlockSpec((1, _BR, _D), lambda t, j: (t, j, 0)),
            pl.BlockSpec((1, _BR, 16), lambda t, j: (t, j, 0)),
            pl.BlockSpec((1, _D), lambda t, j: (0, 0)),
        ],
        out_specs=pl.BlockSpec((_T, _D), lambda t, j: (0, 0)),
    )(acc, yw, dis16, b3)


def _head_body(seq_ref, inw_ref, inb_ref, outw_ref, outb_ref, ln2g_ref,
               ln2b_ref, mw1_ref, mb1_ref, mw2_ref, mb2_ref, lw_ref, lb_ref,
               out_ref):
    seq = seq_ref[...]
    inw = inw_ref[...]
    inb = inb_ref[...]

    def proj(lo):
        return (lax.dot_general(seq, inw[lo:lo + _D, :],
                                (((1,), (1,)), ((), ())),
                                preferred_element_type=jnp.float32)
                + inb[0:1, lo:lo + _D])

    q = proj(0)
    k = proj(_D)
    v = proj(2 * _D)
    hd = _D // _NH
    scale = np.float32(1.0 / np.sqrt(hd).astype(np.float32))
    parts = []
    for h in range(_NH):
        qh = q[:, h * hd:(h + 1) * hd]
        kh = k[:, h * hd:(h + 1) * hd]
        vh = v[:, h * hd:(h + 1) * hd]
        sc = lax.dot_general(qh, kh, (((1,), (1,)), ((), ())),
                             preferred_element_type=jnp.float32) * scale
        m = jnp.max(sc, axis=-1, keepdims=True)
        e = jnp.exp(sc - m)
        attn = e / jnp.sum(e, axis=-1, keepdims=True)
        parts.append(jnp.dot(attn, vh, preferred_element_type=jnp.float32))
    ao = jnp.concatenate(parts, axis=1)
    xa = lax.dot_general(ao, outw_ref[...], (((1,), (1,)), ((), ())),
                         preferred_element_type=jnp.float32) + outb_ref[...]
    h2 = jax.nn.relu(
        lax.dot_general(xa, mw1_ref[...], (((1,), (1,)), ((), ())),
                        preferred_element_type=jnp.float32) + mb1_ref[...])
    h2 = lax.dot_general(h2, mw2_ref[...], (((1,), (1,)), ((), ())),
                         preferred_element_type=jnp.float32) + mb2_ref[...]
    xa = xa + h2
    mean = jnp.mean(xa, axis=-1, keepdims=True)
    var = jnp.mean((xa - mean) ** 2, axis=-1, keepdims=True)
    xa = (xa - mean) / jnp.sqrt(var + 1e-5) * ln2g_ref[...] + ln2b_ref[...]
    xr = jax.nn.relu(xa)
    pooled = jnp.sum(xr, axis=0, keepdims=True)
    out_ref[...] = (lax.dot_general(pooled, lw_ref[...],
                                    (((1,), (1,)), ((), ())),
                                    preferred_element_type=jnp.float32)
                    + lb_ref[...])


def _tc_head(seq, in_w, in_b, out_w, out_b, ln2_g, ln2_b, mw1, mb1, mw2, mb2,
             lw_pad, lb_pad):
    return pl.pallas_call(
        _head_body,
        out_shape=jax.ShapeDtypeStruct((1, _D), jnp.float32),
    )(seq, in_w, in_b, out_w, out_b, ln2_g, ln2_b, mw1, mb1, mw2, mb2,
      lw_pad, lb_pad)


@jax.jit
def kernel(x, edge_index, W1, b1, W2, b2, W3, b3, in_w, in_b, out_w, out_b,
           ln1_g, ln1_b, ln2_g, ln2_b, mw1, mb1, mw2, mb2, lw, lb):
    src = edge_index[:, 0, :]
    dst = edge_index[:, 1, :]
    pad = _EPAD - _E
    src_pad = jnp.concatenate(
        [src, jnp.zeros((_T, pad), jnp.int32)], axis=1)
    dst_pad = jnp.concatenate(
        [dst, jnp.full((_T, pad), _N, jnp.int32)], axis=1)
    src_off = src_pad + (jnp.arange(_T, dtype=jnp.int32) * _N)[:, None]
    srcr = src_off.reshape(_T, _NSUB, _NSUP, _G, _CH)
    dstr = dst_pad.reshape(_T, _NSUB, _NSUP, _G, _CH)

    zeros128 = jnp.zeros((_ZROWS, _D), jnp.float32)
    ones128 = jnp.ones((_CH, _D), jnp.float32)

    deg = _sc_deg(dstr, ones128, zeros128)

    yw1, dis16 = _tc_prescale(x, W1, deg)
    acc1 = _sc_segsum(yw1.reshape(_T * _N, _D), srcr, dstr, zeros128)
    yw2 = _tc_layer(acc1, yw1, dis16, W2, b1.reshape(1, _D))
    acc2 = _sc_segsum(yw2.reshape(_T * _N, _D), srcr, dstr, zeros128)
    yw3 = _tc_layer(acc2, yw2, dis16, W3, b2.reshape(1, _D))
    acc3 = _sc_segsum(yw3.reshape(_T * _N, _D), srcr, dstr, zeros128)
    seq = _tc_final(acc3, yw3, dis16, b3.reshape(1, _D))

    lw_pad = jnp.zeros((_D, _D), jnp.float32).at[: _C, :].set(lw)
    lb_pad = jnp.zeros((1, _D), jnp.float32).at[0, : _C].set(lb)
    out = _tc_head(seq, in_w, in_b.reshape(1, 3 * _D), out_w,
                   out_b.reshape(1, _D), ln2_g.reshape(1, _D),
                   ln2_b.reshape(1, _D), mw1, mb1.reshape(1, _FF), mw2,
                   mb2.reshape(1, _D), lw_pad, lb_pad)
    return out[0, : _C]


# speedup vs baseline: 12.9694x; 1.4446x over previous
import functools
import jax
import jax.numpy as jnp
import numpy as np
from jax import lax
from jax.experimental import pallas as pl
from jax.experimental.pallas import tpu as pltpu
from jax.experimental.pallas import tpu_sc as plsc

_T, _N, _E, _D, _NH, _FF, _C = 8, 10000, 320000, 128, 4, 512, 10
_NSC = 2
_NSUB = 16
_CH = 40
_PH = 4
_G = 2 * _PH
_NSUP = 63
_CPT = _NSUP * _G
_EPT = _CPT * _CH
_EPAD = _NSUB * _EPT
_NACC = 10112
_ZROWS = 632
_TSC = _T // _NSC


def _segsum_body(table, srcr, dstr, zeros_hbm, out, x0, x1, x2,
                 rA, rB, acc, gA, gB, sA, sB, i0, i1, i2):
    idx = [x0, x1, x2]
    isem = [i0, i1, i2]
    c = lax.axis_index("c")
    sid = lax.axis_index("s")

    def gfire(buf, q, xref, p, sem):
        pltpu.async_copy(table.at[xref.at[p]], buf.at[pl.ds(q * _CH, _CH)],
                         sem)

    def gwait(buf, q, sem):
        pltpu.make_async_copy(table.at[x0.at[0]],
                              buf.at[pl.ds(q * _CH, _CH)], sem).wait()

    def sfire(buf, q, xref, p, sem):
        pltpu.async_copy(buf.at[pl.ds(q * _CH, _CH)], acc.at[xref.at[_G + p]],
                         sem, add=True)

    def swait(buf, q, sem):
        pltpu.make_async_copy(buf.at[pl.ds(q * _CH, _CH)], acc.at[x0.at[_G]],
                              sem).wait()

    def load_idx(t, g, xref, sem=None):
        if sem is None:
            pltpu.sync_copy(srcr.at[t, sid, g], xref.at[pl.ds(0, _G)])
            pltpu.sync_copy(dstr.at[t, sid, g], xref.at[pl.ds(_G, _G)])
        else:
            pltpu.async_copy(srcr.at[t, sid, g], xref.at[pl.ds(0, _G)], sem)
            pltpu.async_copy(dstr.at[t, sid, g], xref.at[pl.ds(_G, _G)], sem)

    def wait_idx(t, xref, sem):
        pltpu.make_async_copy(srcr.at[t, sid, 0], xref.at[pl.ds(0, _G)],
                              sem).wait()
        pltpu.make_async_copy(dstr.at[t, sid, 0], xref.at[pl.ds(_G, _G)],
                              sem).wait()

    for tt in range(_TSC):
        t = c * _TSC + tt
        pltpu.sync_copy(zeros_hbm, acc.at[pl.ds(sid * _ZROWS, _ZROWS)])
        load_idx(t, 0, x0)
        load_idx(t, 1, x1, i1)
        plsc.subcore_barrier()

        for q in range(_PH):
            gfire(rA, q, x0, q, gA)

        def giter(gg, _):
            for u in range(3):
                g = 3 * gg + u
                xcur = idx[u]
                xnxt = idx[(u + 1) % 3]

                for q in range(_PH):
                    gwait(rA, q, gA)
                for q in range(_PH):
                    sfire(rA, q, xcur, q, sA)
                if u == 0:
                    @pl.when(gg > 0)
                    def _():
                        for q in range(_PH):
                            swait(rB, q, sB)
                else:
                    for q in range(_PH):
                        swait(rB, q, sB)

                @pl.when(g + 2 < _NSUP)
                def _():
                    load_idx(t, g + 2, idx[(u + 2) % 3], isem[(u + 2) % 3])

                for q in range(_PH):
                    gfire(rB, q, xcur, _PH + q, gB)

                for q in range(_PH):
                    gwait(rB, q, gB)
                for q in range(_PH):
                    sfire(rB, q, xcur, _PH + q, sB)
                for q in range(_PH):
                    swait(rA, q, sA)

                @pl.when(g + 1 < _NSUP)
                def _():
                    wait_idx(t, xnxt, isem[(u + 1) % 3])
                    for q in range(_PH):
                        gfire(rA, q, xnxt, q, gA)
            return 0

        lax.fori_loop(0, _NSUP // 3, giter, 0)
        for q in range(_PH):
            swait(rB, q, sB)
        plsc.subcore_barrier()
        pltpu.sync_copy(acc.at[pl.ds(sid * _ZROWS, _ZROWS)],
                        out.at[t, pl.ds(sid * _ZROWS, _ZROWS)])
        plsc.subcore_barrier()


def _sc_segsum(table_flat, srcr, dstr, zeros_hbm):
    mesh = plsc.VectorSubcoreMesh(core_axis_name="c", subcore_axis_name="s")
    return pl.kernel(
        _segsum_body,
        out_type=jax.ShapeDtypeStruct((_T, _NACC, _D), jnp.float32),
        mesh=mesh,
        scratch_types=(
            [pltpu.VMEM((2 * _G, _CH), jnp.int32) for _ in range(3)]
            + [pltpu.VMEM((_PH * _CH, _D), jnp.float32) for _ in range(2)]
            + [pltpu.VMEM_SHARED((_NACC, _D), jnp.float32)]
            + [pltpu.SemaphoreType.DMA for _ in range(7)]
        ),
    )(table_flat, srcr, dstr, zeros_hbm)


def _deg_body(dstr, ones_hbm, zeros_hbm, out, didx, ones_v, acc, ssem):
    c = lax.axis_index("c")
    s = lax.axis_index("s")
    pltpu.sync_copy(ones_hbm, ones_v)
    for tt in range(_TSC):
        t = c * _TSC + tt
        pltpu.sync_copy(zeros_hbm, acc.at[pl.ds(s * _ZROWS, _ZROWS)])
        plsc.subcore_barrier()

        def super_step(g, _):
            pltpu.sync_copy(dstr.at[t, s, g], didx)
            for j in range(_G):
                pltpu.async_copy(ones_v, acc.at[didx.at[j]], ssem, add=True)
            for j in range(_G):
                pltpu.make_async_copy(ones_v, acc.at[didx.at[j]],
                                      ssem).wait()
            return 0

        lax.fori_loop(0, _NSUP, super_step, 0)
        plsc.subcore_barrier()
        pltpu.sync_copy(acc.at[pl.ds(s * _ZROWS, _ZROWS)],
                        out.at[t, pl.ds(s * _ZROWS, _ZROWS)])
        plsc.subcore_barrier()


def _sc_deg(dstr, ones_hbm, zeros_hbm):
    mesh = plsc.VectorSubcoreMesh(core_axis_name="c", subcore_axis_name="s")
    return pl.kernel(
        _deg_body,
        out_type=jax.ShapeDtypeStruct((_T, _NACC, _D), jnp.float32),
        mesh=mesh,
        scratch_types=[
            pltpu.VMEM((_G, _CH), jnp.int32),
            pltpu.VMEM((_CH, _D), jnp.float32),
            pltpu.VMEM_SHARED((_NACC, _D), jnp.float32),
            pltpu.SemaphoreType.DMA,
        ],
    )(dstr, ones_hbm, zeros_hbm)


_BR = 1000


def _prescale_body(x_ref, w_ref, deg_ref, out_ref, dis_ref):
    dis = lax.rsqrt(deg_ref[0, :, 0:1] + 1.0)
    out_ref[0] = dis * jnp.dot(x_ref[0], w_ref[...],
                               preferred_element_type=jnp.float32)
    dis_ref[0] = jnp.broadcast_to(dis, (_BR, 16))


def _tc_prescale(x, w, deg):
    return pl.pallas_call(
        _prescale_body,
        out_shape=[
            jax.ShapeDtypeStruct((_T, _N, _D), jnp.float32),
            jax.ShapeDtypeStruct((_T, _N, 16), jnp.float32),
        ],
        grid=(_T, _N // _BR),
        in_specs=[
            pl.BlockSpec((1, _BR, _D), lambda t, j: (t, j, 0)),
            pl.BlockSpec((_D, _D), lambda t, j: (0, 0)),
            pl.BlockSpec((1, _BR, _D), lambda t, j: (t, j, 0)),
        ],
        out_specs=[
            pl.BlockSpec((1, _BR, _D), lambda t, j: (t, j, 0)),
            pl.BlockSpec((1, _BR, 16), lambda t, j: (t, j, 0)),
        ],
    )(x, w, deg)


def _layer_body(acc_ref, yw_ref, dis16_ref, w_ref, b_ref, out_ref):
    dis = dis16_ref[0, :, 0:1]
    h = jnp.tanh(dis * (acc_ref[0] + yw_ref[0]) + b_ref[...])
    out_ref[0] = dis * jnp.dot(h, w_ref[...],
                               preferred_element_type=jnp.float32)


def _tc_layer(acc, yw, dis16, w_next, b_prev):
    return pl.pallas_call(
        _layer_body,
        out_shape=jax.ShapeDtypeStruct((_T, _N, _D), jnp.float32),
        grid=(_T, _N // _BR),
        in_specs=[
            pl.BlockSpec((1, _BR, _D), lambda t, j: (t, j, 0)),
            pl.BlockSpec((1, _BR, _D), lambda t, j: (t, j, 0)),
            pl.BlockSpec((1, _BR, 16), lambda t, j: (t, j, 0)),
            pl.BlockSpec((_D, _D), lambda t, j: (0, 0)),
            pl.BlockSpec((1, _D), lambda t, j: (0, 0)),
        ],
        out_specs=pl.BlockSpec((1, _BR, _D), lambda t, j: (t, j, 0)),
    )(acc, yw, dis16, w_next, b_prev)


def _final_body(acc_ref, yw_ref, dis16_ref, b_ref, out_ref):
    t = pl.program_id(0)
    j = pl.program_id(1)
    dis = dis16_ref[0, :, 0:1]
    h = jnp.tanh(dis * (acc_ref[0] + yw_ref[0]) + b_ref[...])
    part = jnp.sum(h, axis=0, keepdims=True)

    @pl.when(j == 0)
    def _():
        out_ref[pl.ds(t, 1), :] = part

    @pl.when(j > 0)
    def _():
        out_ref[pl.ds(t, 1), :] = out_ref[pl.ds(t, 1), :] + part


def _tc_final(acc, yw, dis16, b3):
    return pl.pallas_call(
        _final_body,
        out_shape=jax.ShapeDtypeStruct((_T, _D), jnp.float32),
        grid=(_T, _N // _BR),
        in_specs=[
            pl.BlockSpec((1, _BR, _D), lambda t, j: (t, j, 0)),
            pl.BlockSpec((1, _BR, _D), lambda t, j: (t, j, 0)),
            pl.BlockSpec((1, _BR, 16), lambda t, j: (t, j, 0)),
            pl.BlockSpec((1, _D), lambda t, j: (0, 0)),
        ],
        out_specs=pl.BlockSpec((_T, _D), lambda t, j: (0, 0)),
    )(acc, yw, dis16, b3)


def _head_body(seq_ref, inw_ref, inb_ref, outw_ref, outb_ref, ln2g_ref,
               ln2b_ref, mw1_ref, mb1_ref, mw2_ref, mb2_ref, lw_ref, lb_ref,
               out_ref):
    seq = seq_ref[...]
    inw = inw_ref[...]
    inb = inb_ref[...]

    def proj(lo):
        return (lax.dot_general(seq, inw[lo:lo + _D, :],
                                (((1,), (1,)), ((), ())),
                                preferred_element_type=jnp.float32)
                + inb[0:1, lo:lo + _D])

    q = proj(0)
    k = proj(_D)
    v = proj(2 * _D)
    hd = _D // _NH
    scale = np.float32(1.0 / np.sqrt(hd).astype(np.float32))
    parts = []
    for h in range(_NH):
        qh = q[:, h * hd:(h + 1) * hd]
        kh = k[:, h * hd:(h + 1) * hd]
        vh = v[:, h * hd:(h + 1) * hd]
        sc = lax.dot_general(qh, kh, (((1,), (1,)), ((), ())),
                             preferred_element_type=jnp.float32) * scale
        m = jnp.max(sc, axis=-1, keepdims=True)
        e = jnp.exp(sc - m)
        attn = e / jnp.sum(e, axis=-1, keepdims=True)
        parts.append(jnp.dot(attn, vh, preferred_element_type=jnp.float32))
    ao = jnp.concatenate(parts, axis=1)
    xa = lax.dot_general(ao, outw_ref[...], (((1,), (1,)), ((), ())),
                         preferred_element_type=jnp.float32) + outb_ref[...]
    h2 = jax.nn.relu(
        lax.dot_general(xa, mw1_ref[...], (((1,), (1,)), ((), ())),
                        preferred_element_type=jnp.float32) + mb1_ref[...])
    h2 = lax.dot_general(h2, mw2_ref[...], (((1,), (1,)), ((), ())),
                         preferred_element_type=jnp.float32) + mb2_ref[...]
    xa = xa + h2
    mean = jnp.mean(xa, axis=-1, keepdims=True)
    var = jnp.mean((xa - mean) ** 2, axis=-1, keepdims=True)
    xa = (xa - mean) / jnp.sqrt(var + 1e-5) * ln2g_ref[...] + ln2b_ref[...]
    xr = jax.nn.relu(xa)
    pooled = jnp.sum(xr, axis=0, keepdims=True)
    out_ref[...] = (lax.dot_general(pooled, lw_ref[...],
                                    (((1,), (1,)), ((), ())),
                                    preferred_element_type=jnp.float32)
                    + lb_ref[...])


def _tc_head(seq, in_w, in_b, out_w, out_b, ln2_g, ln2_b, mw1, mb1, mw2, mb2,
             lw_pad, lb_pad):
    return pl.pallas_call(
        _head_body,
        out_shape=jax.ShapeDtypeStruct((1, _D), jnp.float32),
    )(seq, in_w, in_b, out_w, out_b, ln2_g, ln2_b, mw1, mb1, mw2, mb2,
      lw_pad, lb_pad)


@jax.jit
def kernel(x, edge_index, W1, b1, W2, b2, W3, b3, in_w, in_b, out_w, out_b,
           ln1_g, ln1_b, ln2_g, ln2_b, mw1, mb1, mw2, mb2, lw, lb):
    src = edge_index[:, 0, :]
    dst = edge_index[:, 1, :]
    pad = _EPAD - _E
    src_pad = jnp.concatenate(
        [src, jnp.zeros((_T, pad), jnp.int32)], axis=1)
    dst_pad = jnp.concatenate(
        [dst, jnp.full((_T, pad), _N, jnp.int32)], axis=1)
    src_off = src_pad + (jnp.arange(_T, dtype=jnp.int32) * _N)[:, None]
    srcr = src_off.reshape(_T, _NSUB, _NSUP, _G, _CH)
    dstr = dst_pad.reshape(_T, _NSUB, _NSUP, _G, _CH)

    zeros128 = jnp.zeros((_ZROWS, _D), jnp.float32)
    ones128 = jnp.ones((_CH, _D), jnp.float32)

    deg = _sc_deg(dstr, ones128, zeros128)

    yw1, dis16 = _tc_prescale(x, W1, deg)
    acc1 = _sc_segsum(yw1.reshape(_T * _N, _D), srcr, dstr, zeros128)
    yw2 = _tc_layer(acc1, yw1, dis16, W2, b1.reshape(1, _D))
    acc2 = _sc_segsum(yw2.reshape(_T * _N, _D), srcr, dstr, zeros128)
    yw3 = _tc_layer(acc2, yw2, dis16, W3, b2.reshape(1, _D))
    acc3 = _sc_segsum(yw3.reshape(_T * _N, _D), srcr, dstr, zeros128)
    seq = _tc_final(acc3, yw3, dis16, b3.reshape(1, _D))

    lw_pad = jnp.zeros((_D, _D), jnp.float32).at[: _C, :].set(lw)
    lb_pad = jnp.zeros((1, _D), jnp.float32).at[0, : _C].set(lb)
    out = _tc_head(seq, in_w, in_b.reshape(1, 3 * _D), out_w,
                   out_b.reshape(1, _D), ln2_g.reshape(1, _D),
                   ln2_b.reshape(1, _D), mw1, mb1.reshape(1, _FF), mw2,
                   mb2.reshape(1, _D), lw_pad, lb_pad)
    return out[0, : _C]


# speedup vs baseline: 13.0596x; 1.0070x over previous
import functools
import jax
import jax.numpy as jnp
import numpy as np
from jax import lax
from jax.experimental import pallas as pl
from jax.experimental.pallas import tpu as pltpu
from jax.experimental.pallas import tpu_sc as plsc

_T, _N, _E, _D, _NH, _FF, _C = 8, 10000, 320000, 128, 4, 512, 10
_NSC = 2
_NSUB = 16
_CH = 112
_G = 6
_NSUP = 30
_CPT = _NSUP * _G
_EPT = _CPT * _CH
_EPAD = _NSUB * _EPT
_NACC = 10112
_ZROWS = 632
_TSC = _T // _NSC


def _segsum_body(table, srcr, dstr, zeros_hbm, out, x0, x1, x2,
                 r0, r1, r2, acc, g0, g1, g2, s0, s1, s2, i0, i1, i2):
    idx = [x0, x1, x2]
    isem = [i0, i1, i2]
    rows = [r0, r1, r2]
    gs = [g0, g1, g2]
    ss = [s0, s1, s2]
    c = lax.axis_index("c")
    sid = lax.axis_index("s")

    def gfire(b, xref, p, sem):
        pltpu.async_copy(table.at[xref.at[p]], rows[b], sem)

    def gwait(b):
        pltpu.make_async_copy(table.at[x0.at[0]], rows[b], gs[b]).wait()

    def sfire(b, xref, p):
        pltpu.async_copy(rows[b], acc.at[xref.at[_G + p]], ss[b], add=True)

    def swait(b):
        pltpu.make_async_copy(rows[b], acc.at[x0.at[_G]], ss[b]).wait()

    def load_idx(t, g, xref, sem=None):
        if sem is None:
            pltpu.sync_copy(srcr.at[t, sid, g], xref.at[pl.ds(0, _G)])
            pltpu.sync_copy(dstr.at[t, sid, g], xref.at[pl.ds(_G, _G)])
        else:
            pltpu.async_copy(srcr.at[t, sid, g], xref.at[pl.ds(0, _G)], sem)
            pltpu.async_copy(dstr.at[t, sid, g], xref.at[pl.ds(_G, _G)], sem)

    def wait_idx(t, xref, sem):
        pltpu.make_async_copy(srcr.at[t, sid, 0], xref.at[pl.ds(0, _G)],
                              sem).wait()
        pltpu.make_async_copy(dstr.at[t, sid, 0], xref.at[pl.ds(_G, _G)],
                              sem).wait()

    for tt in range(_TSC):
        t = c * _TSC + tt
        pltpu.sync_copy(zeros_hbm, acc.at[pl.ds(sid * _ZROWS, _ZROWS)])
        load_idx(t, 0, x0)
        load_idx(t, 1, x1, i1)
        plsc.subcore_barrier()

        gfire(0, x0, 0, g0)

        def giter(gg, _):
            for u in range(3):
                g = 3 * gg + u
                xc = idx[u]
                xn = idx[(u + 1) % 3]
                for p in range(_G):
                    j = g * _G + p
                    b = p % 3
                    gwait(b)
                    sfire(b, xc, p)

                    @pl.when(j >= 2)
                    def _():
                        swait((b + 1) % 3)

                    if p == 2:
                        @pl.when(g + 2 < _NSUP)
                        def _():
                            load_idx(t, g + 2, idx[(u + 2) % 3],
                                     isem[(u + 2) % 3])
                    if p < _G - 1:
                        gfire((b + 1) % 3, xc, p + 1, gs[(b + 1) % 3])
                    else:
                        @pl.when(j + 1 < _CPT)
                        def _():
                            wait_idx(t, xn, isem[(u + 1) % 3])
                            gfire((b + 1) % 3, xn, 0, gs[(b + 1) % 3])
            return 0

        lax.fori_loop(0, _NSUP // 3, giter, 0)
        swait((_CPT - 2) % 3)
        swait((_CPT - 1) % 3)
        plsc.subcore_barrier()
        pltpu.sync_copy(acc.at[pl.ds(sid * _ZROWS, _ZROWS)],
                        out.at[t, pl.ds(sid * _ZROWS, _ZROWS)])
        plsc.subcore_barrier()


def _sc_segsum(table_flat, srcr, dstr, zeros_hbm):
    mesh = plsc.VectorSubcoreMesh(core_axis_name="c", subcore_axis_name="s")
    return pl.kernel(
        _segsum_body,
        out_type=jax.ShapeDtypeStruct((_T, _NACC, _D), jnp.float32),
        mesh=mesh,
        scratch_types=(
            [pltpu.VMEM((2 * _G, _CH), jnp.int32) for _ in range(3)]
            + [pltpu.VMEM((_CH, _D), jnp.float32) for _ in range(3)]
            + [pltpu.VMEM_SHARED((_NACC, _D), jnp.float32)]
            + [pltpu.SemaphoreType.DMA for _ in range(9)]
        ),
    )(table_flat, srcr, dstr, zeros_hbm)


def _deg_body(dstr, ones_hbm, zeros_hbm, out, didx, ones_v, acc, ssem):
    c = lax.axis_index("c")
    s = lax.axis_index("s")
    pltpu.sync_copy(ones_hbm, ones_v)
    for tt in range(_TSC):
        t = c * _TSC + tt
        pltpu.sync_copy(zeros_hbm, acc.at[pl.ds(s * _ZROWS, _ZROWS)])
        plsc.subcore_barrier()

        def super_step(g, _):
            pltpu.sync_copy(dstr.at[t, s, g], didx)
            for j in range(_G):
                pltpu.async_copy(ones_v, acc.at[didx.at[j]], ssem, add=True)
            for j in range(_G):
                pltpu.make_async_copy(ones_v, acc.at[didx.at[j]],
                                      ssem).wait()
            return 0

        lax.fori_loop(0, _NSUP, super_step, 0)
        plsc.subcore_barrier()
        pltpu.sync_copy(acc.at[pl.ds(s * _ZROWS, _ZROWS)],
                        out.at[t, pl.ds(s * _ZROWS, _ZROWS)])
        plsc.subcore_barrier()


def _sc_deg(dstr, ones_hbm, zeros_hbm):
    mesh = plsc.VectorSubcoreMesh(core_axis_name="c", subcore_axis_name="s")
    return pl.kernel(
        _deg_body,
        out_type=jax.ShapeDtypeStruct((_T, _NACC, _D), jnp.float32),
        mesh=mesh,
        scratch_types=[
            pltpu.VMEM((_G, _CH), jnp.int32),
            pltpu.VMEM((_CH, _D), jnp.float32),
            pltpu.VMEM_SHARED((_NACC, _D), jnp.float32),
            pltpu.SemaphoreType.DMA,
        ],
    )(dstr, ones_hbm, zeros_hbm)


_BR = 1000


def _prescale_body(x_ref, w_ref, deg_ref, out_ref, dis_ref):
    dis = lax.rsqrt(deg_ref[0, :, 0:1] + 1.0)
    out_ref[0] = dis * jnp.dot(x_ref[0], w_ref[...],
                               preferred_element_type=jnp.float32)
    dis_ref[0] = jnp.broadcast_to(dis, (_BR, 16))


def _tc_prescale(x, w, deg):
    return pl.pallas_call(
        _prescale_body,
        out_shape=[
            jax.ShapeDtypeStruct((_T, _N, _D), jnp.float32),
            jax.ShapeDtypeStruct((_T, _N, 16), jnp.float32),
        ],
        grid=(_T, _N // _BR),
        in_specs=[
            pl.BlockSpec((1, _BR, _D), lambda t, j: (t, j, 0)),
            pl.BlockSpec((_D, _D), lambda t, j: (0, 0)),
            pl.BlockSpec((1, _BR, _D), lambda t, j: (t, j, 0)),
        ],
        out_specs=[
            pl.BlockSpec((1, _BR, _D), lambda t, j: (t, j, 0)),
            pl.BlockSpec((1, _BR, 16), lambda t, j: (t, j, 0)),
        ],
    )(x, w, deg)


def _layer_body(acc_ref, yw_ref, dis16_ref, w_ref, b_ref, out_ref):
    dis = dis16_ref[0, :, 0:1]
    h = jnp.tanh(dis * (acc_ref[0] + yw_ref[0]) + b_ref[...])
    out_ref[0] = dis * jnp.dot(h, w_ref[...],
                               preferred_element_type=jnp.float32)


def _tc_layer(acc, yw, dis16, w_next, b_prev):
    return pl.pallas_call(
        _layer_body,
        out_shape=jax.ShapeDtypeStruct((_T, _N, _D), jnp.float32),
        grid=(_T, _N // _BR),
        in_specs=[
            pl.BlockSpec((1, _BR, _D), lambda t, j: (t, j, 0)),
            pl.BlockSpec((1, _BR, _D), lambda t, j: (t, j, 0)),
            pl.BlockSpec((1, _BR, 16), lambda t, j: (t, j, 0)),
            pl.BlockSpec((_D, _D), lambda t, j: (0, 0)),
            pl.BlockSpec((1, _D), lambda t, j: (0, 0)),
        ],
        out_specs=pl.BlockSpec((1, _BR, _D), lambda t, j: (t, j, 0)),
    )(acc, yw, dis16, w_next, b_prev)


def _final_body(acc_ref, yw_ref, dis16_ref, b_ref, out_ref):
    t = pl.program_id(0)
    j = pl.program_id(1)
    dis = dis16_ref[0, :, 0:1]
    h = jnp.tanh(dis * (acc_ref[0] + yw_ref[0]) + b_ref[...])
    part = jnp.sum(h, axis=0, keepdims=True)

    @pl.when(j == 0)
    def _():
        out_ref[pl.ds(t, 1), :] = part

    @pl.when(j > 0)
    def _():
        out_ref[pl.ds(t, 1), :] = out_ref[pl.ds(t, 1), :] + part


def _tc_final(acc, yw, dis16, b3):
    return pl.pallas_call(
        _final_body,
        out_shape=jax.ShapeDtypeStruct((_T, _D), jnp.float32),
        grid=(_T, _N // _BR),
        in_specs=[
            pl.BlockSpec((1, _BR, _D), lambda t, j: (t, j, 0)),
            pl.BlockSpec((1, _BR, _D), lambda t, j: (t, j, 0)),
            pl.BlockSpec((1, _BR, 16), lambda t, j: (t, j, 0)),
            pl.BlockSpec((1, _D), lambda t, j: (0, 0)),
        ],
        out_specs=pl.BlockSpec((_T, _D), lambda t, j: (0, 0)),
    )(acc, yw, dis16, b3)


def _head_body(seq_ref, inw_ref, inb_ref, outw_ref, outb_ref, ln2g_ref,
               ln2b_ref, mw1_ref, mb1_ref, mw2_ref, mb2_ref, lw_ref, lb_ref,
               out_ref):
    seq = seq_ref[...]
    inw = inw_ref[...]
    inb = inb_ref[...]

    def proj(lo):
        return (lax.dot_general(seq, inw[lo:lo + _D, :],
                                (((1,), (1,)), ((), ())),
                                preferred_element_type=jnp.float32)
                + inb[0:1, lo:lo + _D])

    q = proj(0)
    k = proj(_D)
    v = proj(2 * _D)
    hd = _D // _NH
    scale = np.float32(1.0 / np.sqrt(hd).astype(np.float32))
    parts = []
    for h in range(_NH):
        qh = q[:, h * hd:(h + 1) * hd]
        kh = k[:, h * hd:(h + 1) * hd]
        vh = v[:, h * hd:(h + 1) * hd]
        sc = lax.dot_general(qh, kh, (((1,), (1,)), ((), ())),
                             preferred_element_type=jnp.float32) * scale
        m = jnp.max(sc, axis=-1, keepdims=True)
        e = jnp.exp(sc - m)
        attn = e / jnp.sum(e, axis=-1, keepdims=True)
        parts.append(jnp.dot(attn, vh, preferred_element_type=jnp.float32))
    ao = jnp.concatenate(parts, axis=1)
    xa = lax.dot_general(ao, outw_ref[...], (((1,), (1,)), ((), ())),
                         preferred_element_type=jnp.float32) + outb_ref[...]
    h2 = jax.nn.relu(
        lax.dot_general(xa, mw1_ref[...], (((1,), (1,)), ((), ())),
                        preferred_element_type=jnp.float32) + mb1_ref[...])
    h2 = lax.dot_general(h2, mw2_ref[...], (((1,), (1,)), ((), ())),
                         preferred_element_type=jnp.float32) + mb2_ref[...]
    xa = xa + h2
    mean = jnp.mean(xa, axis=-1, keepdims=True)
    var = jnp.mean((xa - mean) ** 2, axis=-1, keepdims=True)
    xa = (xa - mean) / jnp.sqrt(var + 1e-5) * ln2g_ref[...] + ln2b_ref[...]
    xr = jax.nn.relu(xa)
    pooled = jnp.sum(xr, axis=0, keepdims=True)
    out_ref[...] = (lax.dot_general(pooled, lw_ref[...],
                                    (((1,), (1,)), ((), ())),
                                    preferred_element_type=jnp.float32)
                    + lb_ref[...])


def _tc_head(seq, in_w, in_b, out_w, out_b, ln2_g, ln2_b, mw1, mb1, mw2, mb2,
             lw_pad, lb_pad):
    return pl.pallas_call(
        _head_body,
        out_shape=jax.ShapeDtypeStruct((1, _D), jnp.float32),
    )(seq, in_w, in_b, out_w, out_b, ln2_g, ln2_b, mw1, mb1, mw2, mb2,
      lw_pad, lb_pad)


@jax.jit
def kernel(x, edge_index, W1, b1, W2, b2, W3, b3, in_w, in_b, out_w, out_b,
           ln1_g, ln1_b, ln2_g, ln2_b, mw1, mb1, mw2, mb2, lw, lb):
    src = edge_index[:, 0, :]
    dst = edge_index[:, 1, :]
    pad = _EPAD - _E
    src_pad = jnp.concatenate(
        [src, jnp.zeros((_T, pad), jnp.int32)], axis=1)
    dst_pad = jnp.concatenate(
        [dst, jnp.full((_T, pad), _N, jnp.int32)], axis=1)
    src_off = src_pad + (jnp.arange(_T, dtype=jnp.int32) * _N)[:, None]
    srcr = src_off.reshape(_T, _NSUB, _NSUP, _G, _CH)
    dstr = dst_pad.reshape(_T, _NSUB, _NSUP, _G, _CH)

    zeros128 = jnp.zeros((_ZROWS, _D), jnp.float32)
    ones128 = jnp.ones((_CH, _D), jnp.float32)

    deg = _sc_deg(dstr, ones128, zeros128)

    yw1, dis16 = _tc_prescale(x, W1, deg)
    acc1 = _sc_segsum(yw1.reshape(_T * _N, _D), srcr, dstr, zeros128)
    yw2 = _tc_layer(acc1, yw1, dis16, W2, b1.reshape(1, _D))
    acc2 = _sc_segsum(yw2.reshape(_T * _N, _D), srcr, dstr, zeros128)
    yw3 = _tc_layer(acc2, yw2, dis16, W3, b2.reshape(1, _D))
    acc3 = _sc_segsum(yw3.reshape(_T * _N, _D), srcr, dstr, zeros128)
    seq = _tc_final(acc3, yw3, dis16, b3.reshape(1, _D))

    lw_pad = jnp.zeros((_D, _D), jnp.float32).at[: _C, :].set(lw)
    lb_pad = jnp.zeros((1, _D), jnp.float32).at[0, : _C].set(lb)
    out = _tc_head(seq, in_w, in_b.reshape(1, 3 * _D), out_w,
                   out_b.reshape(1, _D), ln2_g.reshape(1, _D),
                   ln2_b.reshape(1, _D), mw1, mb1.reshape(1, _FF), mw2,
                   mb2.reshape(1, _D), lw_pad, lb_pad)
    return out[0, : _C]
